# baseline (device time: 44442 ns/iter reference)
import jax
import jax.numpy as jnp
from jax import lax
from jax.experimental import pallas as pl
from jax.experimental.pallas import tpu as pltpu

N_DEV = 4


def kernel(Q, K, V):
    b, sq, h, d = Q.shape
    _, sk, _, _ = K.shape
    scale = d ** -0.5

    def body(q_ref, k_ref, v_ref, out_ref, comm_ref, send_sems, recv_sems):
        my_pos = lax.axis_index("i")
        left = (my_pos + N_DEV - 1) % N_DEV
        right = (my_pos + 1) % N_DEV

        barrier_sem = pltpu.get_barrier_semaphore()
        for nbr in (left, right):
            pl.semaphore_signal(
                barrier_sem, inc=1,
                device_id=(nbr,), device_id_type=pl.DeviceIdType.MESH,
            )
        pl.semaphore_wait(barrier_sem, 2)

        q = q_ref[:, 0, :, :]
        k = k_ref[...]
        v = v_ref[...]
        s = jnp.sum(q[:, None, :, :] * k, axis=-1) * scale
        m = jnp.max(s, axis=1)
        p = jnp.exp(s - m[:, None, :])
        l = jnp.sum(p, axis=1)
        o = jnp.sum(p[..., None] * v, axis=1)

        comm_ref[0, :, :, 0:d] = o
        comm_ref[0, :, :, d:d + 1] = m[:, :, None]
        comm_ref[0, :, :, d + 1:d + 2] = l[:, :, None]

        o_run = o
        m_run = m[:, :, None]
        l_run = l[:, :, None]

        for hop in range(N_DEV - 1):
            send_slot = hop % 2
            recv_slot = (hop + 1) % 2
            rdma = pltpu.make_async_remote_copy(
                src_ref=comm_ref.at[send_slot],
                dst_ref=comm_ref.at[recv_slot],
                send_sem=send_sems.at[hop],
                recv_sem=recv_sems.at[hop],
                device_id=(right,),
                device_id_type=pl.DeviceIdType.MESH,
            )
            rdma.start()
            rdma.wait()

            o_in = comm_ref[recv_slot, :, :, 0:d]
            m_in = comm_ref[recv_slot, :, :, d:d + 1]
            l_in = comm_ref[recv_slot, :, :, d + 1:d + 2]
            m_new = jnp.maximum(m_run, m_in)
            alpha = jnp.exp(m_run - m_new)
            beta = jnp.exp(m_in - m_new)
            o_run = o_run * alpha + o_in * beta
            l_run = l_run * alpha + l_in * beta
            m_run = m_new

        out_ref[:, 0, :, :] = o_run / l_run

    return pl.pallas_call(
        body,
        out_shape=jax.ShapeDtypeStruct((b, sq, h, d), jnp.float32),
        in_specs=[pl.BlockSpec(memory_space=pltpu.VMEM)] * 3,
        out_specs=pl.BlockSpec(memory_space=pltpu.VMEM),
        scratch_shapes=[
            pltpu.VMEM((2, b, h, 128), jnp.float32),
            pltpu.SemaphoreType.DMA((N_DEV - 1,)),
            pltpu.SemaphoreType.DMA((N_DEV - 1,)),
        ],
        compiler_params=pltpu.CompilerParams(collective_id=0),
    )(Q, K, V)


# device time: 41829 ns/iter; 1.0625x vs baseline; 1.0625x over previous
import jax
import jax.numpy as jnp
from jax import lax
from jax.experimental import pallas as pl
from jax.experimental.pallas import tpu as pltpu

N_DEV = 4


def kernel(Q, K, V):
    b, sq, h, d = Q.shape
    _, sk, _, _ = K.shape
    scale = d ** -0.5

    def body(q_ref, k_ref, v_ref, out_ref, comm_ref, send_sems, recv_sems):
        my_pos = lax.axis_index("i")
        left = (my_pos + N_DEV - 1) % N_DEV
        right = (my_pos + 1) % N_DEV

        barrier_sem = pltpu.get_barrier_semaphore()
        for nbr in (left, right):
            pl.semaphore_signal(
                barrier_sem, inc=1,
                device_id=(nbr,), device_id_type=pl.DeviceIdType.MESH,
            )
        pl.semaphore_wait(barrier_sem, 2)

        bh = b * h
        q2 = q_ref[:, 0, :, :].reshape(bh, d)
        kt = jnp.transpose(k_ref[...], (0, 2, 1, 3)).reshape(bh, sk, d)
        vt = jnp.transpose(v_ref[...], (0, 2, 1, 3)).reshape(bh, sk, d)
        s = lax.dot_general(
            q2, kt,
            dimension_numbers=(((1,), (2,)), ((0,), (0,))),
            preferred_element_type=jnp.float32,
        ) * scale
        m = jnp.max(s, axis=-1, keepdims=True)
        p = jnp.exp(s - m)
        l = jnp.sum(p, axis=-1, keepdims=True)
        o = lax.dot_general(
            p, vt,
            dimension_numbers=(((1,), (1,)), ((0,), (0,))),
            preferred_element_type=jnp.float32,
        )

        comm_ref[0, :, 0:d] = o
        comm_ref[0, :, d:d + 1] = m
        comm_ref[0, :, d + 1:d + 2] = l

        o_run = o
        m_run = m
        l_run = l

        for hop in range(N_DEV - 1):
            send_slot = hop % 2
            recv_slot = (hop + 1) % 2
            rdma = pltpu.make_async_remote_copy(
                src_ref=comm_ref.at[send_slot],
                dst_ref=comm_ref.at[recv_slot],
                send_sem=send_sems.at[hop],
                recv_sem=recv_sems.at[hop],
                device_id=(right,),
                device_id_type=pl.DeviceIdType.MESH,
            )
            rdma.start()
            rdma.wait()

            o_in = comm_ref[recv_slot, :, 0:d]
            m_in = comm_ref[recv_slot, :, d:d + 1]
            l_in = comm_ref[recv_slot, :, d + 1:d + 2]
            m_new = jnp.maximum(m_run, m_in)
            alpha = jnp.exp(m_run - m_new)
            beta = jnp.exp(m_in - m_new)
            o_run = o_run * alpha + o_in * beta
            l_run = l_run * alpha + l_in * beta
            m_run = m_new

        out_ref[:, 0, :, :] = (o_run / l_run).reshape(b, h, d)

    return pl.pallas_call(
        body,
        out_shape=jax.ShapeDtypeStruct((b, sq, h, d), jnp.float32),
        in_specs=[pl.BlockSpec(memory_space=pltpu.VMEM)] * 3,
        out_specs=pl.BlockSpec(memory_space=pltpu.VMEM),
        scratch_shapes=[
            pltpu.VMEM((2, b * h, 128), jnp.float32),
            pltpu.SemaphoreType.DMA((N_DEV - 1,)),
            pltpu.SemaphoreType.DMA((N_DEV - 1,)),
        ],
        compiler_params=pltpu.CompilerParams(collective_id=0),
    )(Q, K, V)


# device time: 26829 ns/iter; 1.6565x vs baseline; 1.5591x over previous
import jax
import jax.numpy as jnp
from jax import lax
from jax.experimental import pallas as pl
from jax.experimental.pallas import tpu as pltpu

N_DEV = 4
GRID = 4


def kernel(Q, K, V):
    b, sq, h, d = Q.shape
    _, sk, _, _ = K.shape
    bh = b * h
    C = sk // GRID
    scale = d ** -0.5
    Kt = jnp.transpose(K, (0, 2, 3, 1)).reshape(bh, d, sk)
    Vt = jnp.transpose(V, (0, 2, 3, 1)).reshape(bh, d, sk)


    def body(q_ref, k_ref, v_ref, out_ref,
             o_acc, m_acc, l_acc, comm_ref, send_sems, recv_sems):
        pi = pl.program_id(0)

        @pl.when(pi == 0)
        def _init():
            m_acc[...] = jnp.full((bh, 1), -1e30, jnp.float32)
            l_acc[...] = jnp.zeros((bh, 1), jnp.float32)
            o_acc[...] = jnp.zeros((bh, d), jnp.float32)

        q2 = q_ref[:, 0, :, :].reshape(bh, d) * scale
        s = lax.dot_general(
            q2, k_ref[...],
            dimension_numbers=(((1,), (1,)), ((0,), (0,))),
            preferred_element_type=jnp.float32,
        )
        m_prev = m_acc[...]
        m_new = jnp.maximum(m_prev, jnp.max(s, axis=-1, keepdims=True))
        alpha = jnp.exp(m_prev - m_new)
        p = jnp.exp(s - m_new)
        o_c = lax.dot_general(
            v_ref[...], p,
            dimension_numbers=(((2,), (1,)), ((0,), (0,))),
            preferred_element_type=jnp.float32,
        )
        m_acc[...] = m_new
        l_acc[...] = l_acc[...] * alpha + jnp.sum(p, axis=-1, keepdims=True)
        o_acc[...] = o_acc[...] * alpha + o_c

        @pl.when(pi == GRID - 1)
        def _comm():
            my_pos = lax.axis_index("i")
            left = (my_pos + N_DEV - 1) % N_DEV
            right = (my_pos + 1) % N_DEV

            barrier_sem = pltpu.get_barrier_semaphore()
            for nbr in (left, right):
                pl.semaphore_signal(
                    barrier_sem, inc=1,
                    device_id=(nbr,), device_id_type=pl.DeviceIdType.MESH,
                )
            pl.semaphore_wait(barrier_sem, 2)

            comm_ref[0, :, 0:d] = o_acc[...]
            comm_ref[0, :, d:d + 1] = m_acc[...]
            comm_ref[0, :, d + 1:d + 2] = l_acc[...]

            o_run = o_acc[...]
            m_run = m_acc[...]
            l_run = l_acc[...]

            for hop in range(N_DEV - 1):
                send_slot = hop % 2
                recv_slot = (hop + 1) % 2
                rdma = pltpu.make_async_remote_copy(
                    src_ref=comm_ref.at[send_slot],
                    dst_ref=comm_ref.at[recv_slot],
                    send_sem=send_sems.at[hop],
                    recv_sem=recv_sems.at[hop],
                    device_id=(right,),
                    device_id_type=pl.DeviceIdType.MESH,
                )
                rdma.start()
                rdma.wait()

                o_in = comm_ref[recv_slot, :, 0:d]
                m_in = comm_ref[recv_slot, :, d:d + 1]
                l_in = comm_ref[recv_slot, :, d + 1:d + 2]
                m_nxt = jnp.maximum(m_run, m_in)
                a = jnp.exp(m_run - m_nxt)
                bta = jnp.exp(m_in - m_nxt)
                o_run = o_run * a + o_in * bta
                l_run = l_run * a + l_in * bta
                m_run = m_nxt

            out_ref[:, 0, :, :] = (o_run / l_run).reshape(b, h, d)

    return pl.pallas_call(
        body,
        grid=(GRID,),
        out_shape=jax.ShapeDtypeStruct((b, sq, h, d), jnp.float32),
        in_specs=[
            pl.BlockSpec((b, sq, h, d), lambda i: (0, 0, 0, 0)),
            pl.BlockSpec((bh, d, C), lambda i: (0, 0, i)),
            pl.BlockSpec((bh, d, C), lambda i: (0, 0, i)),
        ],
        out_specs=pl.BlockSpec((b, sq, h, d), lambda i: (0, 0, 0, 0)),
        scratch_shapes=[
            pltpu.VMEM((bh, d), jnp.float32),
            pltpu.VMEM((bh, 1), jnp.float32),
            pltpu.VMEM((bh, 1), jnp.float32),
            pltpu.VMEM((2, bh, 128), jnp.float32),
            pltpu.SemaphoreType.DMA((N_DEV - 1,)),
            pltpu.SemaphoreType.DMA((N_DEV - 1,)),
        ],
        compiler_params=pltpu.CompilerParams(collective_id=0),
    )(Q, Kt, Vt)


# device time: 25746 ns/iter; 1.7262x vs baseline; 1.0421x over previous
import jax
import jax.numpy as jnp
from jax import lax
from jax.experimental import pallas as pl
from jax.experimental.pallas import tpu as pltpu

N_DEV = 4
N_CHUNK = 4


def kernel(Q, K, V):
    b, sq, h, d = Q.shape
    _, sk, _, _ = K.shape
    bh = b * h
    C = sk // N_CHUNK
    scale = d ** -0.5
    Kt = jnp.transpose(K, (0, 2, 3, 1)).reshape(bh, d, sk)
    Vt = jnp.transpose(V, (0, 2, 3, 1)).reshape(bh, d, sk)


    def body(q_ref, k_ref, v_ref, out_ref,
             kbuf, vbuf, kcp_sems, vcp_sems,
             comm_ref, send_sems, recv_sems):

        def start_chunk(ci):
            slot = ci % 2
            kcp = pltpu.make_async_copy(
                k_ref.at[:, :, ci * C:(ci + 1) * C], kbuf.at[slot],
                kcp_sems.at[slot])
            vcp = pltpu.make_async_copy(
                v_ref.at[:, :, ci * C:(ci + 1) * C], vbuf.at[slot],
                vcp_sems.at[slot])
            kcp.start()
            vcp.start()
            return kcp, vcp

        q2 = q_ref[:, 0, :, :].reshape(bh, d) * scale
        m_run = jnp.full((bh, 1), -1e30, jnp.float32)
        l_run = jnp.zeros((bh, 1), jnp.float32)
        o_run = jnp.zeros((bh, d), jnp.float32)

        cur = start_chunk(0)
        for ci in range(N_CHUNK):
            nxt = start_chunk(ci + 1) if ci + 1 < N_CHUNK else None
            cur[0].wait()
            cur[1].wait()
            slot = ci % 2
            s = lax.dot_general(
                q2, kbuf[slot],
                dimension_numbers=(((1,), (1,)), ((0,), (0,))),
                preferred_element_type=jnp.float32,
            )
            m_new = jnp.maximum(m_run, jnp.max(s, axis=-1, keepdims=True))
            alpha = jnp.exp(m_run - m_new)
            p = jnp.exp(s - m_new)
            o_c = lax.dot_general(
                vbuf[slot], p,
                dimension_numbers=(((2,), (1,)), ((0,), (0,))),
                preferred_element_type=jnp.float32,
            )
            m_run = m_new
            l_run = l_run * alpha + jnp.sum(p, axis=-1, keepdims=True)
            o_run = o_run * alpha + o_c
            cur = nxt

        my_pos = lax.axis_index("i")
        left = (my_pos + N_DEV - 1) % N_DEV
        right = (my_pos + 1) % N_DEV

        barrier_sem = pltpu.get_barrier_semaphore()
        for nbr in (left, right):
            pl.semaphore_signal(
                barrier_sem, inc=1,
                device_id=(nbr,), device_id_type=pl.DeviceIdType.MESH,
            )
        pl.semaphore_wait(barrier_sem, 2)

        comm_ref[0, :, 0:d] = o_run
        comm_ref[0, :, d:d + 1] = m_run
        comm_ref[0, :, d + 1:d + 2] = l_run

        for hop in range(N_DEV - 1):
            send_slot = hop % 2
            recv_slot = (hop + 1) % 2
            rdma = pltpu.make_async_remote_copy(
                src_ref=comm_ref.at[send_slot],
                dst_ref=comm_ref.at[recv_slot],
                send_sem=send_sems.at[hop],
                recv_sem=recv_sems.at[hop],
                device_id=(right,),
                device_id_type=pl.DeviceIdType.MESH,
            )
            rdma.start()
            rdma.wait()

            o_in = comm_ref[recv_slot, :, 0:d]
            m_in = comm_ref[recv_slot, :, d:d + 1]
            l_in = comm_ref[recv_slot, :, d + 1:d + 2]
            m_nxt = jnp.maximum(m_run, m_in)
            a = jnp.exp(m_run - m_nxt)
            bta = jnp.exp(m_in - m_nxt)
            o_run = o_run * a + o_in * bta
            l_run = l_run * a + l_in * bta
            m_run = m_nxt

        out_ref[:, 0, :, :] = (o_run / l_run).reshape(b, h, d)

    return pl.pallas_call(
        body,
        out_shape=jax.ShapeDtypeStruct((b, sq, h, d), jnp.float32),
        in_specs=[
            pl.BlockSpec(memory_space=pltpu.VMEM),
            pl.BlockSpec(memory_space=pl.ANY),
            pl.BlockSpec(memory_space=pl.ANY),
        ],
        out_specs=pl.BlockSpec(memory_space=pltpu.VMEM),
        scratch_shapes=[
            pltpu.VMEM((2, bh, d, C), jnp.float32),
            pltpu.VMEM((2, bh, d, C), jnp.float32),
            pltpu.SemaphoreType.DMA((2,)),
            pltpu.SemaphoreType.DMA((2,)),
            pltpu.VMEM((2, bh, 128), jnp.float32),
            pltpu.SemaphoreType.DMA((N_DEV - 1,)),
            pltpu.SemaphoreType.DMA((N_DEV - 1,)),
        ],
        compiler_params=pltpu.CompilerParams(collective_id=0),
    )(Q, Kt, Vt)


# device time: 18267 ns/iter; 2.4329x vs baseline; 1.4094x over previous
import jax
import jax.numpy as jnp
from jax import lax
from jax.experimental import pallas as pl
from jax.experimental.pallas import tpu as pltpu

N_DEV = 4
N_CHUNK = 4


def kernel(Q, K, V):
    b, sq, h, d = Q.shape
    _, sk, _, _ = K.shape
    bh = b * h
    R = bh // N_CHUNK
    scale = d ** -0.5
    Kt = jnp.transpose(K, (0, 2, 3, 1)).reshape(bh, d, sk)
    Vt = jnp.transpose(V, (0, 2, 3, 1)).reshape(bh, d, sk)


    def body(q_ref, k_ref, v_ref, out_ref,
             kbuf, vbuf, kcp_sems, vcp_sems,
             comm_ref, send_sems, recv_sems):
        my_pos = lax.axis_index("i")

        def start_chunk(ci):
            slot = ci % 2
            rows = pl.ds(ci * R, R)
            kcp = pltpu.make_async_copy(
                k_ref.at[rows], kbuf.at[slot], kcp_sems.at[slot])
            vcp = pltpu.make_async_copy(
                v_ref.at[rows], vbuf.at[slot], vcp_sems.at[slot])
            kcp.start()
            vcp.start()
            return kcp, vcp

        cur = start_chunk(0)

        barrier_sem = pltpu.get_barrier_semaphore()
        for j in range(1, N_DEV):
            pl.semaphore_signal(
                barrier_sem, inc=1,
                device_id=((my_pos + j) % N_DEV,),
                device_id_type=pl.DeviceIdType.MESH,
            )
        pl.semaphore_wait(barrier_sem, N_DEV - 1)

        q2 = q_ref[:, 0, :, :].reshape(bh, d) * scale
        for ci in range(N_CHUNK):
            nxt = start_chunk(ci + 1) if ci + 1 < N_CHUNK else None
            cur[0].wait()
            cur[1].wait()
            slot = ci % 2
            rows = pl.ds(ci * R, R)
            s = lax.dot_general(
                q2[ci * R:(ci + 1) * R], kbuf[slot],
                dimension_numbers=(((1,), (1,)), ((0,), (0,))),
                preferred_element_type=jnp.float32,
            )
            m = jnp.max(s, axis=-1, keepdims=True)
            p = jnp.exp(s - m)
            l = jnp.sum(p, axis=-1, keepdims=True)
            o = lax.dot_general(
                vbuf[slot], p,
                dimension_numbers=(((2,), (1,)), ((0,), (0,))),
                preferred_element_type=jnp.float32,
            )
            comm_ref[0, rows, 0:d] = o
            comm_ref[0, rows, d:d + 1] = m
            comm_ref[0, rows, d + 1:d + 2] = l
            cur = nxt

        rdmas = []
        for j in range(1, N_DEV):
            rdma = pltpu.make_async_remote_copy(
                src_ref=comm_ref.at[0],
                dst_ref=comm_ref.at[N_DEV - j],
                send_sem=send_sems.at[j - 1],
                recv_sem=recv_sems.at[N_DEV - j],
                device_id=((my_pos + j) % N_DEV,),
                device_id_type=pl.DeviceIdType.MESH,
            )
            rdma.start()
            rdmas.append(rdma)

        o_run = comm_ref[0, :, 0:d]
        m_run = comm_ref[0, :, d:d + 1]
        l_run = comm_ref[0, :, d + 1:d + 2]

        for r in (1, 2, 3):
            rdmas[N_DEV - 1 - r].wait_recv()
            o_in = comm_ref[r, :, 0:d]
            m_in = comm_ref[r, :, d:d + 1]
            l_in = comm_ref[r, :, d + 1:d + 2]
            m_nxt = jnp.maximum(m_run, m_in)
            a = jnp.exp(m_run - m_nxt)
            bta = jnp.exp(m_in - m_nxt)
            o_run = o_run * a + o_in * bta
            l_run = l_run * a + l_in * bta
            m_run = m_nxt

        out_ref[:, 0, :, :] = (o_run / l_run).reshape(b, h, d)

        for rdma in rdmas:
            rdma.wait_send()

    return pl.pallas_call(
        body,
        out_shape=jax.ShapeDtypeStruct((b, sq, h, d), jnp.float32),
        in_specs=[
            pl.BlockSpec(memory_space=pltpu.VMEM),
            pl.BlockSpec(memory_space=pl.ANY),
            pl.BlockSpec(memory_space=pl.ANY),
        ],
        out_specs=pl.BlockSpec(memory_space=pltpu.VMEM),
        scratch_shapes=[
            pltpu.VMEM((2, bh // N_CHUNK, d, sk), jnp.float32),
            pltpu.VMEM((2, bh // N_CHUNK, d, sk), jnp.float32),
            pltpu.SemaphoreType.DMA((2,)),
            pltpu.SemaphoreType.DMA((2,)),
            pltpu.VMEM((N_DEV, bh, 128), jnp.float32),
            pltpu.SemaphoreType.DMA((N_DEV - 1,)),
            pltpu.SemaphoreType.DMA((N_DEV,)),
        ],
        compiler_params=pltpu.CompilerParams(collective_id=0),
    )(Q, Kt, Vt)


# device time: 18240 ns/iter; 2.4365x vs baseline; 1.0015x over previous
import jax
import jax.numpy as jnp
from jax import lax
from jax.experimental import pallas as pl
from jax.experimental.pallas import tpu as pltpu

N_DEV = 4
N_CHUNK = 4


def kernel(Q, K, V):
    b, sq, h, d = Q.shape
    _, sk, _, _ = K.shape
    bh = b * h
    R = bh // N_CHUNK
    scale = d ** -0.5
    Kt = jnp.transpose(K, (0, 2, 3, 1)).reshape(bh, d, sk)
    Vt = jnp.transpose(V, (0, 2, 3, 1)).reshape(bh, d, sk)
    Qt = Q.reshape(bh, d)


    def body(q_ref, k_ref, v_ref, out_ref,
             qbuf, kbuf, vbuf, qcp_sem, kcp_sems, vcp_sems,
             comm_ref, send_sems, recv_sems):
        my_pos = lax.axis_index("i")

        def start_chunk(ci):
            slot = ci % 2
            rows = pl.ds(ci * R, R)
            kcp = pltpu.make_async_copy(
                k_ref.at[rows], kbuf.at[slot], kcp_sems.at[slot])
            vcp = pltpu.make_async_copy(
                v_ref.at[rows], vbuf.at[slot], vcp_sems.at[slot])
            kcp.start()
            vcp.start()
            return kcp, vcp

        qcp = pltpu.make_async_copy(q_ref, qbuf, qcp_sem)
        qcp.start()
        cur = start_chunk(0)

        barrier_sem = pltpu.get_barrier_semaphore()
        for j in range(1, N_DEV):
            pl.semaphore_signal(
                barrier_sem, inc=1,
                device_id=((my_pos + j) % N_DEV,),
                device_id_type=pl.DeviceIdType.MESH,
            )
        pl.semaphore_wait(barrier_sem, N_DEV - 1)

        qcp.wait()
        q2 = qbuf[...] * scale
        for ci in range(N_CHUNK):
            nxt = start_chunk(ci + 1) if ci + 1 < N_CHUNK else None
            cur[0].wait()
            cur[1].wait()
            slot = ci % 2
            rows = pl.ds(ci * R, R)
            s = lax.dot_general(
                q2[ci * R:(ci + 1) * R], kbuf[slot],
                dimension_numbers=(((1,), (1,)), ((0,), (0,))),
                preferred_element_type=jnp.float32,
            )
            m = jnp.max(s, axis=-1, keepdims=True)
            p = jnp.exp(s - m)
            l = jnp.sum(p, axis=-1, keepdims=True)
            o = lax.dot_general(
                vbuf[slot], p,
                dimension_numbers=(((2,), (1,)), ((0,), (0,))),
                preferred_element_type=jnp.float32,
            )
            comm_ref[0, rows, 0:d] = o
            comm_ref[0, rows, d:d + 1] = m
            comm_ref[0, rows, d + 1:d + 2] = l
            cur = nxt

        rdmas = []
        for j in range(1, N_DEV):
            rdma = pltpu.make_async_remote_copy(
                src_ref=comm_ref.at[0],
                dst_ref=comm_ref.at[N_DEV - j],
                send_sem=send_sems.at[j - 1],
                recv_sem=recv_sems.at[N_DEV - j],
                device_id=((my_pos + j) % N_DEV,),
                device_id_type=pl.DeviceIdType.MESH,
            )
            rdma.start()
            rdmas.append(rdma)

        o_run = comm_ref[0, :, 0:d]
        m_run = comm_ref[0, :, d:d + 1]
        l_run = comm_ref[0, :, d + 1:d + 2]

        for r in (1, 2, 3):
            rdmas[N_DEV - 1 - r].wait_recv()
            o_in = comm_ref[r, :, 0:d]
            m_in = comm_ref[r, :, d:d + 1]
            l_in = comm_ref[r, :, d + 1:d + 2]
            m_nxt = jnp.maximum(m_run, m_in)
            a = jnp.exp(m_run - m_nxt)
            bta = jnp.exp(m_in - m_nxt)
            o_run = o_run * a + o_in * bta
            l_run = l_run * a + l_in * bta
            m_run = m_nxt

        out_ref[:, 0, :, :] = (o_run / l_run).reshape(b, h, d)

        for rdma in rdmas:
            rdma.wait_send()

    return pl.pallas_call(
        body,
        out_shape=jax.ShapeDtypeStruct((b, sq, h, d), jnp.float32),
        in_specs=[
            pl.BlockSpec(memory_space=pltpu.MemorySpace.HBM),
            pl.BlockSpec(memory_space=pltpu.MemorySpace.HBM),
            pl.BlockSpec(memory_space=pltpu.MemorySpace.HBM),
        ],
        out_specs=pl.BlockSpec(memory_space=pltpu.VMEM),
        scratch_shapes=[
            pltpu.VMEM((bh, d), jnp.float32),
            pltpu.VMEM((2, bh // N_CHUNK, d, sk), jnp.float32),
            pltpu.VMEM((2, bh // N_CHUNK, d, sk), jnp.float32),
            pltpu.SemaphoreType.DMA(()),
            pltpu.SemaphoreType.DMA((2,)),
            pltpu.SemaphoreType.DMA((2,)),
            pltpu.VMEM((N_DEV, bh, 128), jnp.float32),
            pltpu.SemaphoreType.DMA((N_DEV - 1,)),
            pltpu.SemaphoreType.DMA((N_DEV,)),
        ],
        compiler_params=pltpu.CompilerParams(collective_id=0),
    )(Qt, Kt, Vt)


# device time: 12641 ns/iter; 3.5157x vs baseline; 1.4429x over previous
import jax
import jax.numpy as jnp
from jax import lax
from jax.experimental import pallas as pl
from jax.experimental.pallas import tpu as pltpu

N_DEV = 4
N_CHUNK = 4


def kernel(Q, K, V):
    b, sq, h, d = Q.shape
    _, sk, _, _ = K.shape
    bh = b * h
    R = bh // N_CHUNK
    scale = d ** -0.5
    Kt = jnp.transpose(K, (0, 2, 3, 1)).reshape(bh, d, sk)
    Vt = jnp.transpose(V, (0, 2, 3, 1)).reshape(bh, d, sk)
    Qt = Q.reshape(bh, d)
    Kt = pltpu.with_memory_space_constraint(Kt, pltpu.MemorySpace.HBM)
    Vt = pltpu.with_memory_space_constraint(Vt, pltpu.MemorySpace.HBM)
    Qt = pltpu.with_memory_space_constraint(Qt, pltpu.MemorySpace.HBM)


    def body(q_ref, k_ref, v_ref, out_ref,
             qbuf, kbuf, vbuf, qcp_sem, kcp_sems, vcp_sems,
             comm_ref, send_sems, recv_sems):
        my_pos = lax.axis_index("i")

        def start_chunk(ci):
            slot = ci % 2
            rows = pl.ds(ci * R, R)
            kcp = pltpu.make_async_copy(
                k_ref.at[rows], kbuf.at[slot], kcp_sems.at[slot])
            vcp = pltpu.make_async_copy(
                v_ref.at[rows], vbuf.at[slot], vcp_sems.at[slot])
            kcp.start()
            vcp.start()
            return kcp, vcp

        qcp = pltpu.make_async_copy(q_ref, qbuf, qcp_sem)
        qcp.start()
        cur = start_chunk(0)

        barrier_sem = pltpu.get_barrier_semaphore()
        for j in range(1, N_DEV):
            pl.semaphore_signal(
                barrier_sem, inc=1,
                device_id=((my_pos + j) % N_DEV,),
                device_id_type=pl.DeviceIdType.MESH,
            )
        pl.semaphore_wait(barrier_sem, N_DEV - 1)

        qcp.wait()
        q2 = qbuf[...] * scale
        for ci in range(N_CHUNK):
            nxt = start_chunk(ci + 1) if ci + 1 < N_CHUNK else None
            cur[0].wait()
            cur[1].wait()
            slot = ci % 2
            rows = pl.ds(ci * R, R)
            s = lax.dot_general(
                q2[ci * R:(ci + 1) * R], kbuf[slot],
                dimension_numbers=(((1,), (1,)), ((0,), (0,))),
                preferred_element_type=jnp.float32,
            )
            m = jnp.max(s, axis=-1, keepdims=True)
            p = jnp.exp(s - m)
            l = jnp.sum(p, axis=-1, keepdims=True)
            o = lax.dot_general(
                vbuf[slot], p,
                dimension_numbers=(((2,), (1,)), ((0,), (0,))),
                preferred_element_type=jnp.float32,
            )
            comm_ref[0, rows, 0:d] = o
            comm_ref[0, rows, d:d + 1] = m
            comm_ref[0, rows, d + 1:d + 2] = l
            cur = nxt

        rdmas = []
        for j in range(1, N_DEV):
            rdma = pltpu.make_async_remote_copy(
                src_ref=comm_ref.at[0],
                dst_ref=comm_ref.at[N_DEV - j],
                send_sem=send_sems.at[j - 1],
                recv_sem=recv_sems.at[N_DEV - j],
                device_id=((my_pos + j) % N_DEV,),
                device_id_type=pl.DeviceIdType.MESH,
            )
            rdma.start()
            rdmas.append(rdma)

        o_run = comm_ref[0, :, 0:d]
        m_run = comm_ref[0, :, d:d + 1]
        l_run = comm_ref[0, :, d + 1:d + 2]

        for r in (1, 2, 3):
            rdmas[N_DEV - 1 - r].wait_recv()
            o_in = comm_ref[r, :, 0:d]
            m_in = comm_ref[r, :, d:d + 1]
            l_in = comm_ref[r, :, d + 1:d + 2]
            m_nxt = jnp.maximum(m_run, m_in)
            a = jnp.exp(m_run - m_nxt)
            bta = jnp.exp(m_in - m_nxt)
            o_run = o_run * a + o_in * bta
            l_run = l_run * a + l_in * bta
            m_run = m_nxt

        out_ref[:, 0, :, :] = (o_run / l_run).reshape(b, h, d)

        for rdma in rdmas:
            rdma.wait_send()

    return pl.pallas_call(
        body,
        out_shape=jax.ShapeDtypeStruct((b, sq, h, d), jnp.float32),
        in_specs=[
            pl.BlockSpec(memory_space=pltpu.MemorySpace.HBM),
            pl.BlockSpec(memory_space=pltpu.MemorySpace.HBM),
            pl.BlockSpec(memory_space=pltpu.MemorySpace.HBM),
        ],
        out_specs=pl.BlockSpec(memory_space=pltpu.VMEM),
        scratch_shapes=[
            pltpu.VMEM((bh, d), jnp.float32),
            pltpu.VMEM((2, bh // N_CHUNK, d, sk), jnp.float32),
            pltpu.VMEM((2, bh // N_CHUNK, d, sk), jnp.float32),
            pltpu.SemaphoreType.DMA(()),
            pltpu.SemaphoreType.DMA((2,)),
            pltpu.SemaphoreType.DMA((2,)),
            pltpu.VMEM((N_DEV, bh, 128), jnp.float32),
            pltpu.SemaphoreType.DMA((N_DEV - 1,)),
            pltpu.SemaphoreType.DMA((N_DEV,)),
        ],
        compiler_params=pltpu.CompilerParams(collective_id=0),
    )(Qt, Kt, Vt)


# device time: 12120 ns/iter; 3.6668x vs baseline; 1.0430x over previous
import jax
import jax.numpy as jnp
from jax import lax
from jax.experimental import pallas as pl
from jax.experimental.pallas import tpu as pltpu

N_DEV = 4
N_CHUNK = 2


def kernel(Q, K, V):
    b, sq, h, d = Q.shape
    _, sk, _, _ = K.shape
    bh = b * h
    R = bh // N_CHUNK
    scale = d ** -0.5
    Kt = jnp.transpose(K, (0, 2, 3, 1)).reshape(bh, d, sk)
    Vt = jnp.transpose(V, (0, 2, 3, 1)).reshape(bh, d, sk)
    Qt = Q.reshape(bh, d)
    Kt = pltpu.with_memory_space_constraint(Kt, pltpu.MemorySpace.HBM)
    Vt = pltpu.with_memory_space_constraint(Vt, pltpu.MemorySpace.HBM)
    Qt = pltpu.with_memory_space_constraint(Qt, pltpu.MemorySpace.HBM)


    def body(q_ref, k_ref, v_ref, out_ref,
             qbuf, kbuf, vbuf, qcp_sem, kcp_sems, vcp_sems,
             comm_ref, send_sems, recv_sems):
        my_pos = lax.axis_index("i")

        def start_chunk(ci):
            slot = ci % 2
            rows = pl.ds(ci * R, R)
            kcp = pltpu.make_async_copy(
                k_ref.at[rows], kbuf.at[slot], kcp_sems.at[slot])
            vcp = pltpu.make_async_copy(
                v_ref.at[rows], vbuf.at[slot], vcp_sems.at[slot])
            kcp.start()
            vcp.start()
            return kcp, vcp

        qcp = pltpu.make_async_copy(q_ref, qbuf, qcp_sem)
        qcp.start()
        cur = start_chunk(0)

        barrier_sem = pltpu.get_barrier_semaphore()
        for j in range(1, N_DEV):
            pl.semaphore_signal(
                barrier_sem, inc=1,
                device_id=((my_pos + j) % N_DEV,),
                device_id_type=pl.DeviceIdType.MESH,
            )
        pl.semaphore_wait(barrier_sem, N_DEV - 1)

        def issue_round(t):
            rds = []
            for j in range(1, N_DEV):
                rdma = pltpu.make_async_remote_copy(
                    src_ref=comm_ref.at[0, t],
                    dst_ref=comm_ref.at[N_DEV - j, t],
                    send_sem=send_sems.at[t, j - 1],
                    recv_sem=recv_sems.at[t, N_DEV - j],
                    device_id=((my_pos + j) % N_DEV,),
                    device_id_type=pl.DeviceIdType.MESH,
                )
                rdma.start()
                rds.append(rdma)
            return rds

        half = N_CHUNK // 2
        rounds = [None, None]
        qcp.wait()
        q2 = qbuf[...] * scale
        for ci in range(N_CHUNK):
            nxt = start_chunk(ci + 1) if ci + 1 < N_CHUNK else None
            cur[0].wait()
            slot = ci % 2
            t, rr = ci // half, ci % half
            s = lax.dot_general(
                q2[ci * R:(ci + 1) * R], kbuf[slot],
                dimension_numbers=(((1,), (1,)), ((0,), (0,))),
                preferred_element_type=jnp.float32,
            )
            m = jnp.max(s, axis=-1, keepdims=True)
            p = jnp.exp(s - m)
            l = jnp.sum(p, axis=-1, keepdims=True)
            cur[1].wait()
            o = lax.dot_general(
                vbuf[slot], p,
                dimension_numbers=(((2,), (1,)), ((0,), (0,))),
                preferred_element_type=jnp.float32,
            )
            comm_ref[0, t, rr * R:(rr + 1) * R, 0:d] = o
            comm_ref[0, t, rr * R:(rr + 1) * R, d:d + 1] = m
            comm_ref[0, t, rr * R:(rr + 1) * R, d + 1:d + 2] = l
            if ci == half - 1:
                rounds[0] = issue_round(0)
            cur = nxt

        rounds[1] = issue_round(1)

        bt = b // 2
        for t in (0, 1):
            o_run = comm_ref[0, t, :, 0:d]
            m_run = comm_ref[0, t, :, d:d + 1]
            l_run = comm_ref[0, t, :, d + 1:d + 2]
            for r in (1, 2, 3):
                rounds[t][N_DEV - 1 - r].wait_recv()
                o_in = comm_ref[r, t, :, 0:d]
                m_in = comm_ref[r, t, :, d:d + 1]
                l_in = comm_ref[r, t, :, d + 1:d + 2]
                m_nxt = jnp.maximum(m_run, m_in)
                a = jnp.exp(m_run - m_nxt)
                bta = jnp.exp(m_in - m_nxt)
                o_run = o_run * a + o_in * bta
                l_run = l_run * a + l_in * bta
                m_run = m_nxt
            out_ref[t * bt:(t + 1) * bt, 0, :, :] = (
                (o_run / l_run).reshape(bt, h, d))

        for t in (0, 1):
            for rdma in rounds[t]:
                rdma.wait_send()

    return pl.pallas_call(
        body,
        out_shape=jax.ShapeDtypeStruct((b, sq, h, d), jnp.float32),
        in_specs=[
            pl.BlockSpec(memory_space=pltpu.MemorySpace.HBM),
            pl.BlockSpec(memory_space=pltpu.MemorySpace.HBM),
            pl.BlockSpec(memory_space=pltpu.MemorySpace.HBM),
        ],
        out_specs=pl.BlockSpec(memory_space=pltpu.VMEM),
        scratch_shapes=[
            pltpu.VMEM((bh, d), jnp.float32),
            pltpu.VMEM((2, bh // N_CHUNK, d, sk), jnp.float32),
            pltpu.VMEM((2, bh // N_CHUNK, d, sk), jnp.float32),
            pltpu.SemaphoreType.DMA(()),
            pltpu.SemaphoreType.DMA((2,)),
            pltpu.SemaphoreType.DMA((2,)),
            pltpu.VMEM((N_DEV, 2, bh // 2, 128), jnp.float32),
            pltpu.SemaphoreType.DMA((2, N_DEV - 1)),
            pltpu.SemaphoreType.DMA((2, N_DEV)),
        ],
        compiler_params=pltpu.CompilerParams(collective_id=0),
    )(Qt, Kt, Vt)
